# baseline (speedup 1.0000x reference)
import functools

import jax
import jax.numpy as jnp
from jax import lax
from jax.experimental import pallas as pl
from jax.experimental.pallas import tpu as pltpu
from jax.experimental.pallas import tpu_sc as plsc

NC = 2
NS = 16
NW = NC * NS
LANES = 16
CH = 128
NBUF = 8
D_HID = 16
ROWBLK = 1000


def _mesh():
    return plsc.VectorSubcoreMesh(
        core_axis_name="c", subcore_axis_name="s", num_cores=NC, num_subcores=NS
    )


def _chunk_range(cid, sid, c0, c1):
    cnt = jnp.where(cid == 0, c0, c1)
    base = jnp.where(cid == 0, sid * c0, NS * c0 + sid * c1)
    return base, cnt


def _make_deg_kernel(c0, c1, n_pad, rpt):
    @functools.partial(
        pl.kernel,
        out_type=jax.ShapeDtypeStruct((NC, n_pad), jnp.float32),
        mesh=_mesh(),
        scratch_types=[
            pltpu.VMEM((max(c0, c1), CH), jnp.int32),
            pltpu.VMEM((CH,), jnp.float32),
            pltpu.VMEM((CH,), jnp.float32),
            pltpu.VMEM_SHARED((n_pad,), jnp.float32),
        ]
        + [pltpu.SemaphoreType.DMA] * NBUF,
    )
    def deg_kernel(dst_hbm, out_hbm, idx_v, ones_v, zero_v, acc, *sems):
        cid = lax.axis_index("c")
        sid = lax.axis_index("s")
        cbase, cnt = _chunk_range(cid, sid, c0, c1)
        for i in range(CH // LANES):
            ones_v[pl.ds(LANES * i, LANES)] = jnp.full((LANES,), 1.0, jnp.float32)
            zero_v[pl.ds(LANES * i, LANES)] = jnp.zeros((LANES,), jnp.float32)

        @pl.when(cid == 0)
        def _():
            pltpu.sync_copy(dst_hbm.at[pl.ds(cbase, c0)], idx_v.at[pl.ds(0, c0)])

        @pl.when(cid != 0)
        def _():
            pltpu.sync_copy(dst_hbm.at[pl.ds(cbase, c1)], idx_v.at[pl.ds(0, c1)])

        base = sid * rpt
        for t in range(rpt // CH):
            pltpu.sync_copy(zero_v, acc.at[pl.ds(base + t * CH, CH)])
        plsc.subcore_barrier()

        for b in range(NBUF):
            pltpu.async_copy(ones_v, acc.at[idx_v.at[b]], sems[b], add=True)

        def group(g, carry):
            for b in range(NBUF):
                j = g * NBUF + b
                pltpu.make_async_copy(ones_v, acc.at[idx_v.at[j]], sems[b]).wait()

                @pl.when(j + NBUF < cnt)
                def _():
                    pltpu.async_copy(
                        ones_v, acc.at[idx_v.at[j + NBUF]], sems[b], add=True
                    )

            return carry

        lax.fori_loop(0, cnt // NBUF, group, 0)
        plsc.subcore_barrier()
        pltpu.sync_copy(acc.at[pl.ds(base, rpt)], out_hbm.at[cid, pl.ds(base, rpt)])

    return deg_kernel


def _make_agg_kernel(c0, c1, n_pad, rpt):
    cmax = max(c0, c1)

    @functools.partial(
        pl.kernel,
        out_type=jax.ShapeDtypeStruct((NC, n_pad, D_HID), jnp.float32),
        mesh=_mesh(),
        scratch_types=[
            pltpu.VMEM((cmax, CH), jnp.int32),
            pltpu.VMEM((cmax, CH), jnp.int32),
            pltpu.VMEM((NBUF, CH, D_HID), jnp.float32),
            pltpu.VMEM_SHARED((n_pad, D_HID), jnp.float32),
        ]
        + [pltpu.SemaphoreType.DMA] * (2 * NBUF),
        compiler_params=pltpu.CompilerParams(use_tc_tiling_on_sc=False),
        name="gcn_row_agg",
    )
    def agg_kernel(table_hbm, src_hbm, dst_hbm, out_hbm, src_v, dst_v, rows_v,
                   acc, *sems):
        cid = lax.axis_index("c")
        sid = lax.axis_index("s")
        cbase, cnt = _chunk_range(cid, sid, c0, c1)
        base = sid * rpt

        @pl.when(cid == 0)
        def _():
            pltpu.sync_copy(src_hbm.at[pl.ds(cbase, c0)], src_v.at[pl.ds(0, c0)])
            pltpu.sync_copy(dst_hbm.at[pl.ds(cbase, c0)], dst_v.at[pl.ds(0, c0)])

        @pl.when(cid != 0)
        def _():
            pltpu.sync_copy(src_hbm.at[pl.ds(cbase, c1)], src_v.at[pl.ds(0, c1)])
            pltpu.sync_copy(dst_hbm.at[pl.ds(cbase, c1)], dst_v.at[pl.ds(0, c1)])

        def zrow(i, carry):
            rows_v[0, i, :] = jnp.zeros((LANES,), jnp.float32)
            return carry

        lax.fori_loop(0, CH, zrow, 0)
        for t in range(rpt // CH):
            pltpu.sync_copy(rows_v.at[0], acc.at[pl.ds(base + t * CH, CH)])
        plsc.subcore_barrier()

        gsems = sems[:NBUF]
        ssems = sems[NBUF:]

        for b in range(NBUF):
            pltpu.async_copy(table_hbm.at[src_v.at[b]], rows_v.at[b], gsems[b])

        def group(g, carry):
            for b in range(NBUF):
                j = g * NBUF + b
                pltpu.make_async_copy(
                    table_hbm.at[src_v.at[j]], rows_v.at[b], gsems[b]
                ).wait()
                pltpu.async_copy(
                    rows_v.at[b], acc.at[dst_v.at[j]], ssems[b], add=True
                )
            for b in range(NBUF):
                j = g * NBUF + b
                pltpu.make_async_copy(
                    rows_v.at[b], acc.at[dst_v.at[j]], ssems[b]
                ).wait()

                @pl.when(j + NBUF < cnt)
                def _():
                    pltpu.async_copy(
                        table_hbm.at[src_v.at[j + NBUF]], rows_v.at[b], gsems[b]
                    )

            return carry

        lax.fori_loop(0, cnt // NBUF, group, 0)
        plsc.subcore_barrier()
        pltpu.sync_copy(
            acc.at[pl.ds(base, rpt)], out_hbm.at[cid, pl.ds(base, rpt)]
        )

    return agg_kernel


def _tc_a_body(x_ref, w_ref, degp_ref, zp_ref, dinv_ref):
    z = jnp.dot(x_ref[...], w_ref[...], preferred_element_type=jnp.float32)
    deg = 1.0 + degp_ref[0] + degp_ref[1]
    dinv = 1.0 / jnp.sqrt(deg)
    dinv_ref[...] = dinv
    zp_ref[...] = z * dinv


def _tc_a(x, W1, degp):
    n, d_in = x.shape
    grid = n // ROWBLK
    return pl.pallas_call(
        _tc_a_body,
        grid=(grid,),
        in_specs=[
            pl.BlockSpec((ROWBLK, d_in), lambda i: (i, 0)),
            pl.BlockSpec((d_in, D_HID), lambda i: (0, 0)),
            pl.BlockSpec((NC, ROWBLK, 1), lambda i: (0, i, 0)),
        ],
        out_specs=[
            pl.BlockSpec((ROWBLK, D_HID), lambda i: (i, 0)),
            pl.BlockSpec((ROWBLK, 1), lambda i: (i, 0)),
        ],
        out_shape=[
            jax.ShapeDtypeStruct((n, D_HID), jnp.float32),
            jax.ShapeDtypeStruct((n, 1), jnp.float32),
        ],
    )(x, W1, degp)


def _tc_b_body(agg_ref, zp_ref, dinv_ref, b1_ref, out_ref):
    s = agg_ref[0] + agg_ref[1] + zp_ref[...]
    dinv = dinv_ref[...]
    h1 = jnp.maximum(dinv * s + b1_ref[...], 0.0)
    out_ref[...] = h1 * dinv


def _tc_b(agg1, zp, dinv, b1):
    n = zp.shape[0]
    grid = n // ROWBLK
    return pl.pallas_call(
        _tc_b_body,
        grid=(grid,),
        in_specs=[
            pl.BlockSpec((NC, ROWBLK, D_HID), lambda i: (0, i, 0)),
            pl.BlockSpec((ROWBLK, D_HID), lambda i: (i, 0)),
            pl.BlockSpec((ROWBLK, 1), lambda i: (i, 0)),
            pl.BlockSpec((1, D_HID), lambda i: (0, 0)),
        ],
        out_specs=pl.BlockSpec((ROWBLK, D_HID), lambda i: (i, 0)),
        out_shape=jax.ShapeDtypeStruct((n, D_HID), jnp.float32),
    )(agg1, zp, dinv, b1)


def _tc_c_body(agg_ref, h1p_ref, dinv_ref, w2_ref, b2_ref, out_ref):
    s = agg_ref[0] + agg_ref[1] + h1p_ref[...]
    pre = dinv_ref[...] * s
    h2 = jnp.dot(pre, w2_ref[...], preferred_element_type=jnp.float32)
    h2 = h2 + b2_ref[...]
    m = jnp.max(h2, axis=1, keepdims=True)
    e = jnp.exp(h2 - m)
    lse = jnp.log(jnp.sum(e, axis=1, keepdims=True))
    out_ref[...] = h2 - m - lse


def _tc_c(agg2, h1p, dinv, W2, b2):
    n = h1p.shape[0]
    n_cls = W2.shape[1]
    grid = n // ROWBLK
    return pl.pallas_call(
        _tc_c_body,
        grid=(grid,),
        in_specs=[
            pl.BlockSpec((NC, ROWBLK, D_HID), lambda i: (0, i, 0)),
            pl.BlockSpec((ROWBLK, D_HID), lambda i: (i, 0)),
            pl.BlockSpec((ROWBLK, 1), lambda i: (i, 0)),
            pl.BlockSpec((D_HID, n_cls), lambda i: (0, 0)),
            pl.BlockSpec((1, n_cls), lambda i: (0, 0)),
        ],
        out_specs=pl.BlockSpec((ROWBLK, n_cls), lambda i: (i, 0)),
        out_shape=jax.ShapeDtypeStruct((n, n_cls), jnp.float32),
    )(agg2, h1p, dinv, W2, b2)


F0 = 0.70


def kernel(x, edge_index, W1, b1, W2, b2):
    n = x.shape[0]
    e = edge_index.shape[1]

    tch = -(-e // CH)
    nch_min = -(-tch // NS)
    c0 = -(-int(nch_min * F0) // NBUF) * NBUF
    c1 = -(-(nch_min - c0) // NBUF) * NBUF
    tot_ch = NS * (c0 + c1)
    e_pad = tot_ch * CH
    n_pad = -(-(n + 1) // (NS * CH)) * (NS * CH)
    rpt = n_pad // NS

    src = edge_index[0].astype(jnp.int32)
    dst = edge_index[1].astype(jnp.int32)
    pad = e_pad - e
    src2 = jnp.concatenate([jnp.zeros((pad,), jnp.int32), src]).reshape(tot_ch, CH)
    dst2 = jnp.concatenate([jnp.full((pad,), n, jnp.int32), dst]).reshape(tot_ch, CH)

    degp = _make_deg_kernel(c0, c1, n_pad, rpt)(dst2)

    zp, dinv = _tc_a(x, W1, degp.reshape(NC, n_pad, 1))
    agg = _make_agg_kernel(c0, c1, n_pad, rpt)
    agg1 = agg(zp, src2, dst2)
    h1p = _tc_b(agg1, zp, dinv, b1.reshape(1, D_HID))
    agg2 = agg(h1p, src2, dst2)
    return _tc_c(agg2, h1p, dinv, W2, b2.reshape(1, -1))


# speedup vs baseline: 1.0986x; 1.0986x over previous
import functools

import jax
import jax.numpy as jnp
from jax import lax
from jax.experimental import pallas as pl
from jax.experimental.pallas import tpu as pltpu
from jax.experimental.pallas import tpu_sc as plsc

NC = 2
NS = 16
NW = NC * NS
LANES = 16
CH = 128
NBUF = 8
D_HID = 16
ROWBLK = 1000


def _mesh():
    return plsc.VectorSubcoreMesh(
        core_axis_name="c", subcore_axis_name="s", num_cores=NC, num_subcores=NS
    )


def _chunk_range(cid, sid, c0, c1):
    cnt = jnp.where(cid == 0, c0, c1)
    base = jnp.where(cid == 0, sid * c0, NS * c0 + sid * c1)
    return base, cnt


def _make_deg_kernel(c0, c1, n_pad, rpt):
    @functools.partial(
        pl.kernel,
        out_type=jax.ShapeDtypeStruct((NC, n_pad), jnp.float32),
        mesh=_mesh(),
        scratch_types=[
            pltpu.VMEM((max(c0, c1), CH), jnp.int32),
            pltpu.VMEM((CH,), jnp.float32),
            pltpu.VMEM((CH,), jnp.float32),
            pltpu.VMEM_SHARED((n_pad,), jnp.float32),
        ]
        + [pltpu.SemaphoreType.DMA] * NBUF,
    )
    def deg_kernel(dst_hbm, out_hbm, idx_v, ones_v, zero_v, acc, *sems):
        cid = lax.axis_index("c")
        sid = lax.axis_index("s")
        cbase, cnt = _chunk_range(cid, sid, c0, c1)
        for i in range(CH // LANES):
            ones_v[pl.ds(LANES * i, LANES)] = jnp.full((LANES,), 1.0, jnp.float32)
            zero_v[pl.ds(LANES * i, LANES)] = jnp.zeros((LANES,), jnp.float32)

        @pl.when(cid == 0)
        def _():
            pltpu.sync_copy(dst_hbm.at[pl.ds(cbase, c0)], idx_v.at[pl.ds(0, c0)])

        @pl.when(cid != 0)
        def _():
            pltpu.sync_copy(dst_hbm.at[pl.ds(cbase, c1)], idx_v.at[pl.ds(0, c1)])

        base = sid * rpt
        for t in range(rpt // CH):
            pltpu.sync_copy(zero_v, acc.at[pl.ds(base + t * CH, CH)])
        plsc.subcore_barrier()

        for b in range(NBUF):
            pltpu.async_copy(ones_v, acc.at[idx_v.at[b]], sems[b], add=True)

        def group(g, carry):
            for b in range(NBUF):
                j = g * NBUF + b
                pltpu.make_async_copy(ones_v, acc.at[idx_v.at[j]], sems[b]).wait()

                @pl.when(j + NBUF < cnt)
                def _():
                    pltpu.async_copy(
                        ones_v, acc.at[idx_v.at[j + NBUF]], sems[b], add=True
                    )

            return carry

        lax.fori_loop(0, cnt // NBUF, group, 0)
        plsc.subcore_barrier()
        pltpu.sync_copy(acc.at[pl.ds(base, rpt)], out_hbm.at[cid, pl.ds(base, rpt)])

    return deg_kernel


def _make_agg_kernel(c0, c1, n_pad, rpt):
    cmax = max(c0, c1)

    @functools.partial(
        pl.kernel,
        out_type=jax.ShapeDtypeStruct((NC, n_pad, D_HID), jnp.float32),
        mesh=_mesh(),
        scratch_types=[
            pltpu.VMEM((cmax, CH), jnp.int32),
            pltpu.VMEM((cmax, CH), jnp.int32),
            pltpu.VMEM((NBUF, CH, D_HID), jnp.float32),
            pltpu.VMEM_SHARED((n_pad, D_HID), jnp.float32),
        ]
        + [pltpu.SemaphoreType.DMA] * (2 * NBUF),
        compiler_params=pltpu.CompilerParams(use_tc_tiling_on_sc=False),
        name="gcn_row_agg",
    )
    def agg_kernel(table_hbm, src_hbm, dst_hbm, out_hbm, src_v, dst_v, rows_v,
                   acc, *sems):
        cid = lax.axis_index("c")
        sid = lax.axis_index("s")
        cbase, cnt = _chunk_range(cid, sid, c0, c1)
        base = sid * rpt

        @pl.when(cid == 0)
        def _():
            pltpu.sync_copy(src_hbm.at[pl.ds(cbase, c0)], src_v.at[pl.ds(0, c0)])
            pltpu.sync_copy(dst_hbm.at[pl.ds(cbase, c0)], dst_v.at[pl.ds(0, c0)])

        @pl.when(cid != 0)
        def _():
            pltpu.sync_copy(src_hbm.at[pl.ds(cbase, c1)], src_v.at[pl.ds(0, c1)])
            pltpu.sync_copy(dst_hbm.at[pl.ds(cbase, c1)], dst_v.at[pl.ds(0, c1)])

        def zrow(i, carry):
            rows_v[0, i, :] = jnp.zeros((LANES,), jnp.float32)
            return carry

        lax.fori_loop(0, CH, zrow, 0)
        for t in range(rpt // CH):
            pltpu.sync_copy(rows_v.at[0], acc.at[pl.ds(base + t * CH, CH)])
        plsc.subcore_barrier()

        gsems = sems[:NBUF]
        ssems = sems[NBUF:]

        for b in range(NBUF):
            pltpu.async_copy(table_hbm.at[src_v.at[b]], rows_v.at[b], gsems[b])

        def group(g, carry):
            for b in range(NBUF):
                j = g * NBUF + b
                pltpu.make_async_copy(
                    table_hbm.at[src_v.at[j]], rows_v.at[b], gsems[b]
                ).wait()
                pltpu.async_copy(
                    rows_v.at[b], acc.at[dst_v.at[j]], ssems[b], add=True
                )
            for b in range(NBUF):
                j = g * NBUF + b
                pltpu.make_async_copy(
                    rows_v.at[b], acc.at[dst_v.at[j]], ssems[b]
                ).wait()

                @pl.when(j + NBUF < cnt)
                def _():
                    pltpu.async_copy(
                        table_hbm.at[src_v.at[j + NBUF]], rows_v.at[b], gsems[b]
                    )

            return carry

        lax.fori_loop(0, cnt // NBUF, group, 0)
        plsc.subcore_barrier()
        pltpu.sync_copy(
            acc.at[pl.ds(base, rpt)], out_hbm.at[cid, pl.ds(base, rpt)]
        )

    return agg_kernel


def _tc_a_body(x_ref, w_ref, degp_ref, zp_ref, dinv_ref):
    z = jnp.dot(x_ref[...], w_ref[...], preferred_element_type=jnp.float32)
    deg = 1.0 + degp_ref[0] + degp_ref[1]
    dinv = 1.0 / jnp.sqrt(deg)
    dinv_ref[...] = dinv
    zp_ref[...] = z * dinv


def _tc_a(x, W1, degp):
    n, d_in = x.shape
    grid = n // ROWBLK
    return pl.pallas_call(
        _tc_a_body,
        grid=(grid,),
        in_specs=[
            pl.BlockSpec((ROWBLK, d_in), lambda i: (i, 0)),
            pl.BlockSpec((d_in, D_HID), lambda i: (0, 0)),
            pl.BlockSpec((NC, ROWBLK, 1), lambda i: (0, i, 0)),
        ],
        out_specs=[
            pl.BlockSpec((ROWBLK, D_HID), lambda i: (i, 0)),
            pl.BlockSpec((ROWBLK, 1), lambda i: (i, 0)),
        ],
        out_shape=[
            jax.ShapeDtypeStruct((n, D_HID), jnp.float32),
            jax.ShapeDtypeStruct((n, 1), jnp.float32),
        ],
    )(x, W1, degp)


def _tc_b_body(agg_ref, zp_ref, dinv_ref, b1_ref, out_ref):
    s = agg_ref[0] + agg_ref[1] + zp_ref[...]
    dinv = dinv_ref[...]
    h1 = jnp.maximum(dinv * s + b1_ref[...], 0.0)
    out_ref[...] = h1 * dinv


def _tc_b(agg1, zp, dinv, b1):
    n = zp.shape[0]
    grid = n // ROWBLK
    return pl.pallas_call(
        _tc_b_body,
        grid=(grid,),
        in_specs=[
            pl.BlockSpec((NC, ROWBLK, D_HID), lambda i: (0, i, 0)),
            pl.BlockSpec((ROWBLK, D_HID), lambda i: (i, 0)),
            pl.BlockSpec((ROWBLK, 1), lambda i: (i, 0)),
            pl.BlockSpec((1, D_HID), lambda i: (0, 0)),
        ],
        out_specs=pl.BlockSpec((ROWBLK, D_HID), lambda i: (i, 0)),
        out_shape=jax.ShapeDtypeStruct((n, D_HID), jnp.float32),
    )(agg1, zp, dinv, b1)


def _tc_c_body(agg_ref, h1p_ref, dinv_ref, w2_ref, b2_ref, out_ref):
    s = agg_ref[0] + agg_ref[1] + h1p_ref[...]
    pre = dinv_ref[...] * s
    h2 = jnp.dot(pre, w2_ref[...], preferred_element_type=jnp.float32)
    h2 = h2 + b2_ref[...]
    m = jnp.max(h2, axis=1, keepdims=True)
    e = jnp.exp(h2 - m)
    lse = jnp.log(jnp.sum(e, axis=1, keepdims=True))
    out_ref[...] = h2 - m - lse


def _tc_c(agg2, h1p, dinv, W2, b2):
    n = h1p.shape[0]
    n_cls = W2.shape[1]
    grid = n // ROWBLK
    return pl.pallas_call(
        _tc_c_body,
        grid=(grid,),
        in_specs=[
            pl.BlockSpec((NC, ROWBLK, D_HID), lambda i: (0, i, 0)),
            pl.BlockSpec((ROWBLK, D_HID), lambda i: (i, 0)),
            pl.BlockSpec((ROWBLK, 1), lambda i: (i, 0)),
            pl.BlockSpec((D_HID, n_cls), lambda i: (0, 0)),
            pl.BlockSpec((1, n_cls), lambda i: (0, 0)),
        ],
        out_specs=pl.BlockSpec((ROWBLK, n_cls), lambda i: (i, 0)),
        out_shape=jax.ShapeDtypeStruct((n, n_cls), jnp.float32),
    )(agg2, h1p, dinv, W2, b2)


F0 = 0.30


def kernel(x, edge_index, W1, b1, W2, b2):
    n = x.shape[0]
    e = edge_index.shape[1]

    tch = -(-e // CH)
    nch_min = -(-tch // NS)
    c0 = -(-int(nch_min * F0) // NBUF) * NBUF
    c1 = -(-(nch_min - c0) // NBUF) * NBUF
    tot_ch = NS * (c0 + c1)
    e_pad = tot_ch * CH
    n_pad = -(-(n + 1) // (NS * CH)) * (NS * CH)
    rpt = n_pad // NS

    src = edge_index[0].astype(jnp.int32)
    dst = edge_index[1].astype(jnp.int32)
    pad = e_pad - e
    src2 = jnp.concatenate([src, jnp.zeros((pad,), jnp.int32)]).reshape(tot_ch, CH)
    dst2 = jnp.concatenate([dst, jnp.full((pad,), n, jnp.int32)]).reshape(tot_ch, CH)

    degp = _make_deg_kernel(c0, c1, n_pad, rpt)(dst2)

    zp, dinv = _tc_a(x, W1, degp.reshape(NC, n_pad, 1))
    agg = _make_agg_kernel(c0, c1, n_pad, rpt)
    agg1 = agg(zp, src2, dst2)
    h1p = _tc_b(agg1, zp, dinv, b1.reshape(1, D_HID))
    agg2 = agg(h1p, src2, dst2)
    return _tc_c(agg2, h1p, dinv, W2, b2.reshape(1, -1))


# speedup vs baseline: 1.7223x; 1.5678x over previous
import functools

import jax
import jax.numpy as jnp
from jax import lax
from jax.experimental import pallas as pl
from jax.experimental.pallas import tpu as pltpu
from jax.experimental.pallas import tpu_sc as plsc

NC = 2
NS = 16
NW = NC * NS
LANES = 16
CH = 128
NBUF = 4
D_HID = 16
ROWBLK = 1000


def _mesh():
    return plsc.VectorSubcoreMesh(
        core_axis_name="c", subcore_axis_name="s", num_cores=NC, num_subcores=NS
    )


def _worker_range(cid, sid, gper, grem):
    w = sid * NC + cid
    extra = (w < grem).astype(jnp.int32)
    cnt = (gper + extra) * NBUF
    base = (gper * w + jnp.minimum(w, grem)) * NBUF
    return base, cnt, w


def _make_deg_kernel(gper, grem, n_pad, rpt):
    cmax = (gper + (1 if grem else 0)) * NBUF

    @functools.partial(
        pl.kernel,
        out_type=jax.ShapeDtypeStruct((NC, n_pad), jnp.float32),
        mesh=_mesh(),
        scratch_types=[
            pltpu.VMEM((cmax, CH), jnp.int32),
            pltpu.VMEM((CH,), jnp.float32),
            pltpu.VMEM((CH,), jnp.float32),
            pltpu.VMEM_SHARED((n_pad,), jnp.float32),
        ]
        + [pltpu.SemaphoreType.DMA] * NBUF,
        compiler_params=pltpu.CompilerParams(use_tc_tiling_on_sc=False),
        name="gcn_deg",
    )
    def deg_kernel(ei_hbm, out_hbm, idx_v, ones_v, zero_v, acc, *sems):
        cid = lax.axis_index("c")
        sid = lax.axis_index("s")
        cbase, cnt, w = _worker_range(cid, sid, gper, grem)
        for i in range(CH // LANES):
            ones_v[pl.ds(LANES * i, LANES)] = jnp.full((LANES,), 1.0, jnp.float32)
            zero_v[pl.ds(LANES * i, LANES)] = jnp.zeros((LANES,), jnp.float32)

        @pl.when(w < grem)
        def _():
            pltpu.sync_copy(ei_hbm.at[1, pl.ds(cbase, cmax)], idx_v)

        @pl.when(w >= grem)
        def _():
            pltpu.sync_copy(
                ei_hbm.at[1, pl.ds(cbase, gper * NBUF)],
                idx_v.at[pl.ds(0, gper * NBUF)],
            )

        base = sid * rpt
        for t in range(rpt // CH):
            pltpu.sync_copy(zero_v, acc.at[pl.ds(base + t * CH, CH)])
        plsc.subcore_barrier()

        for b in range(NBUF):
            pltpu.async_copy(ones_v, acc.at[idx_v.at[b]], sems[b], add=True)

        def group(g, carry):
            for b in range(NBUF):
                j = g * NBUF + b
                pltpu.make_async_copy(ones_v, acc.at[idx_v.at[j]], sems[b]).wait()

                @pl.when(j + NBUF < cnt)
                def _():
                    pltpu.async_copy(
                        ones_v, acc.at[idx_v.at[j + NBUF]], sems[b], add=True
                    )

            return carry

        lax.fori_loop(0, cnt // NBUF, group, 0)
        plsc.subcore_barrier()
        pltpu.sync_copy(acc.at[pl.ds(base, rpt)], out_hbm.at[cid, pl.ds(base, rpt)])

    return deg_kernel


def _make_agg_kernel(gper, grem, n_pad, rpt):
    cmax = (gper + (1 if grem else 0)) * NBUF

    @functools.partial(
        pl.kernel,
        out_type=jax.ShapeDtypeStruct((NC, n_pad, D_HID), jnp.float32),
        mesh=_mesh(),
        scratch_types=[
            pltpu.VMEM((cmax, CH), jnp.int32),
            pltpu.VMEM((cmax, CH), jnp.int32),
            pltpu.VMEM((NBUF, CH, D_HID), jnp.float32),
            pltpu.VMEM_SHARED((n_pad, D_HID), jnp.float32),
        ]
        + [pltpu.SemaphoreType.DMA] * (2 * NBUF),
        compiler_params=pltpu.CompilerParams(use_tc_tiling_on_sc=False),
        name="gcn_row_agg",
    )
    def agg_kernel(table_hbm, ei_hbm, out_hbm, src_v, dst_v, rows_v, acc, *sems):
        cid = lax.axis_index("c")
        sid = lax.axis_index("s")
        cbase, cnt, w = _worker_range(cid, sid, gper, grem)
        base = sid * rpt

        @pl.when(w < grem)
        def _():
            pltpu.sync_copy(ei_hbm.at[0, pl.ds(cbase, cmax)], src_v)
            pltpu.sync_copy(ei_hbm.at[1, pl.ds(cbase, cmax)], dst_v)

        @pl.when(w >= grem)
        def _():
            pltpu.sync_copy(
                ei_hbm.at[0, pl.ds(cbase, gper * NBUF)],
                src_v.at[pl.ds(0, gper * NBUF)],
            )
            pltpu.sync_copy(
                ei_hbm.at[1, pl.ds(cbase, gper * NBUF)],
                dst_v.at[pl.ds(0, gper * NBUF)],
            )

        def zrow(i, carry):
            rows_v[0, i, :] = jnp.zeros((LANES,), jnp.float32)
            return carry

        lax.fori_loop(0, CH, zrow, 0)
        for t in range(rpt // CH):
            pltpu.sync_copy(rows_v.at[0], acc.at[pl.ds(base + t * CH, CH)])
        plsc.subcore_barrier()

        gsems = sems[:NBUF]
        ssems = sems[NBUF:]

        for b in range(NBUF):
            pltpu.async_copy(table_hbm.at[src_v.at[b]], rows_v.at[b], gsems[b])

        def group(g, carry):
            for b in range(NBUF):
                j = g * NBUF + b
                pltpu.make_async_copy(
                    table_hbm.at[src_v.at[j]], rows_v.at[b], gsems[b]
                ).wait()
                pltpu.async_copy(
                    rows_v.at[b], acc.at[dst_v.at[j]], ssems[b], add=True
                )
            for b in range(NBUF):
                j = g * NBUF + b
                pltpu.make_async_copy(
                    rows_v.at[b], acc.at[dst_v.at[j]], ssems[b]
                ).wait()

                @pl.when(j + NBUF < cnt)
                def _():
                    pltpu.async_copy(
                        table_hbm.at[src_v.at[j + NBUF]], rows_v.at[b], gsems[b]
                    )

            return carry

        lax.fori_loop(0, cnt // NBUF, group, 0)
        plsc.subcore_barrier()
        pltpu.sync_copy(
            acc.at[pl.ds(base, rpt)], out_hbm.at[cid, pl.ds(base, rpt)]
        )

    return agg_kernel


def _tc_a_body(x_ref, w_ref, degp_ref, zp_ref, dinv_ref):
    z = jnp.dot(x_ref[...], w_ref[...], preferred_element_type=jnp.float32)
    deg = 1.0 + degp_ref[0] + degp_ref[1]
    dinv = 1.0 / jnp.sqrt(deg)
    dinv_ref[...] = dinv
    zp_ref[...] = z * dinv


def _tc_a(x, W1, degp):
    n, d_in = x.shape
    grid = n // ROWBLK
    return pl.pallas_call(
        _tc_a_body,
        grid=(grid,),
        in_specs=[
            pl.BlockSpec((ROWBLK, d_in), lambda i: (i, 0)),
            pl.BlockSpec((d_in, D_HID), lambda i: (0, 0)),
            pl.BlockSpec((NC, ROWBLK, 1), lambda i: (0, i, 0)),
        ],
        out_specs=[
            pl.BlockSpec((ROWBLK, D_HID), lambda i: (i, 0)),
            pl.BlockSpec((ROWBLK, 1), lambda i: (i, 0)),
        ],
        out_shape=[
            jax.ShapeDtypeStruct((n, D_HID), jnp.float32),
            jax.ShapeDtypeStruct((n, 1), jnp.float32),
        ],
    )(x, W1, degp)


def _tc_b_body(agg_ref, zp_ref, dinv_ref, b1_ref, out_ref):
    s = agg_ref[0] + agg_ref[1] + zp_ref[...]
    dinv = dinv_ref[...]
    h1 = jnp.maximum(dinv * s + b1_ref[...], 0.0)
    out_ref[...] = h1 * dinv


def _tc_b(agg1, zp, dinv, b1):
    n = zp.shape[0]
    grid = n // ROWBLK
    return pl.pallas_call(
        _tc_b_body,
        grid=(grid,),
        in_specs=[
            pl.BlockSpec((NC, ROWBLK, D_HID), lambda i: (0, i, 0)),
            pl.BlockSpec((ROWBLK, D_HID), lambda i: (i, 0)),
            pl.BlockSpec((ROWBLK, 1), lambda i: (i, 0)),
            pl.BlockSpec((1, D_HID), lambda i: (0, 0)),
        ],
        out_specs=pl.BlockSpec((ROWBLK, D_HID), lambda i: (i, 0)),
        out_shape=jax.ShapeDtypeStruct((n, D_HID), jnp.float32),
    )(agg1, zp, dinv, b1)


def _tc_c_body(agg_ref, h1p_ref, dinv_ref, w2_ref, b2_ref, out_ref):
    s = agg_ref[0] + agg_ref[1] + h1p_ref[...]
    pre = dinv_ref[...] * s
    h2 = jnp.dot(pre, w2_ref[...], preferred_element_type=jnp.float32)
    h2 = h2 + b2_ref[...]
    m = jnp.max(h2, axis=1, keepdims=True)
    e = jnp.exp(h2 - m)
    lse = jnp.log(jnp.sum(e, axis=1, keepdims=True))
    out_ref[...] = h2 - m - lse


def _tc_c(agg2, h1p, dinv, W2, b2):
    n = h1p.shape[0]
    n_cls = W2.shape[1]
    grid = n // ROWBLK
    return pl.pallas_call(
        _tc_c_body,
        grid=(grid,),
        in_specs=[
            pl.BlockSpec((NC, ROWBLK, D_HID), lambda i: (0, i, 0)),
            pl.BlockSpec((ROWBLK, D_HID), lambda i: (i, 0)),
            pl.BlockSpec((ROWBLK, 1), lambda i: (i, 0)),
            pl.BlockSpec((D_HID, n_cls), lambda i: (0, 0)),
            pl.BlockSpec((1, n_cls), lambda i: (0, 0)),
        ],
        out_specs=pl.BlockSpec((ROWBLK, n_cls), lambda i: (i, 0)),
        out_shape=jax.ShapeDtypeStruct((n, n_cls), jnp.float32),
    )(agg2, h1p, dinv, W2, b2)


def kernel(x, edge_index, W1, b1, W2, b2):
    n = x.shape[0]
    e = edge_index.shape[1]

    gtot = -(-e // (CH * NBUF))
    e_pad = gtot * CH * NBUF
    ei = edge_index.astype(jnp.int32)
    if e_pad != e:
        fill = jnp.stack(
            [
                jnp.zeros((e_pad - e,), jnp.int32),
                jnp.full((e_pad - e,), n, jnp.int32),
            ]
        )
        ei = jnp.concatenate([ei, fill], axis=1)
    ei3 = ei.reshape(2, e_pad // CH, CH)

    gper, grem = gtot // NW, gtot % NW
    n_pad = -(-(n + 1) // (NS * CH)) * (NS * CH)
    rpt = n_pad // NS

    degp = _make_deg_kernel(gper, grem, n_pad, rpt)(ei3)

    zp, dinv = _tc_a(x, W1, degp.reshape(NC, n_pad, 1))
    agg = _make_agg_kernel(gper, grem, n_pad, rpt)
    agg1 = agg(zp, ei3)
    h1p = _tc_b(agg1, zp, dinv, b1.reshape(1, D_HID))
    agg2 = agg(h1p, ei3)
    return _tc_c(agg2, h1p, dinv, W2, b2.reshape(1, -1))


# speedup vs baseline: 2.2500x; 1.3064x over previous
import functools

import jax
import jax.numpy as jnp
from jax import lax
from jax.experimental import pallas as pl
from jax.experimental.pallas import tpu as pltpu
from jax.experimental.pallas import tpu_sc as plsc

NC = 2
NS = 16
NW = NC * NS
LANES = 16
CH = 128
NBUF = 4
D_HID = 16
ROWBLK = 1024


def _mesh():
    return plsc.VectorSubcoreMesh(
        core_axis_name="c", subcore_axis_name="s", num_cores=NC, num_subcores=NS
    )


def _worker_range(cid, sid, gper, grem):
    w = sid * NC + cid
    extra = (w < grem).astype(jnp.int32)
    cnt = (gper + extra) * NBUF
    base = (gper * w + jnp.minimum(w, grem)) * NBUF
    return base, cnt, w


def _make_deg_kernel(gper, grem, n_pad, rpt):
    cmax = (gper + (1 if grem else 0)) * NBUF

    @functools.partial(
        pl.kernel,
        out_type=jax.ShapeDtypeStruct((NC, n_pad), jnp.float32),
        mesh=_mesh(),
        scratch_types=[
            pltpu.VMEM((cmax, CH), jnp.int32),
            pltpu.VMEM((CH,), jnp.float32),
            pltpu.VMEM((CH,), jnp.float32),
            pltpu.VMEM_SHARED((n_pad,), jnp.float32),
        ]
        + [pltpu.SemaphoreType.DMA] * NBUF,
        compiler_params=pltpu.CompilerParams(use_tc_tiling_on_sc=False),
        name="gcn_deg",
    )
    def deg_kernel(ei_hbm, out_hbm, idx_v, ones_v, zero_v, acc, *sems):
        cid = lax.axis_index("c")
        sid = lax.axis_index("s")
        cbase, cnt, w = _worker_range(cid, sid, gper, grem)
        for i in range(CH // LANES):
            ones_v[pl.ds(LANES * i, LANES)] = jnp.full((LANES,), 1.0, jnp.float32)
            zero_v[pl.ds(LANES * i, LANES)] = jnp.zeros((LANES,), jnp.float32)

        @pl.when(w < grem)
        def _():
            pltpu.sync_copy(ei_hbm.at[1, pl.ds(cbase, cmax)], idx_v)

        @pl.when(w >= grem)
        def _():
            pltpu.sync_copy(
                ei_hbm.at[1, pl.ds(cbase, gper * NBUF)],
                idx_v.at[pl.ds(0, gper * NBUF)],
            )

        base = sid * rpt
        for t in range(rpt // CH):
            pltpu.sync_copy(zero_v, acc.at[pl.ds(base + t * CH, CH)])
        plsc.subcore_barrier()

        for b in range(NBUF):
            pltpu.async_copy(ones_v, acc.at[idx_v.at[b]], sems[b], add=True)

        def group(g, carry):
            for b in range(NBUF):
                j = g * NBUF + b
                pltpu.make_async_copy(ones_v, acc.at[idx_v.at[j]], sems[b]).wait()

                @pl.when(j + NBUF < cnt)
                def _():
                    pltpu.async_copy(
                        ones_v, acc.at[idx_v.at[j + NBUF]], sems[b], add=True
                    )

            return carry

        lax.fori_loop(0, cnt // NBUF, group, 0)
        plsc.subcore_barrier()
        pltpu.sync_copy(acc.at[pl.ds(base, rpt)], out_hbm.at[cid, pl.ds(base, rpt)])

    return deg_kernel


def _make_agg_kernel(gper, grem, n_pad, rpt):
    cmax = (gper + (1 if grem else 0)) * NBUF

    @functools.partial(
        pl.kernel,
        out_type=jax.ShapeDtypeStruct((NC, n_pad, D_HID), jnp.float32),
        mesh=_mesh(),
        scratch_types=[
            pltpu.VMEM((cmax, CH), jnp.int32),
            pltpu.VMEM((cmax, CH), jnp.int32),
            pltpu.VMEM((NBUF, CH, D_HID), jnp.float32),
            pltpu.VMEM_SHARED((n_pad, D_HID), jnp.float32),
        ]
        + [pltpu.SemaphoreType.DMA] * (2 * NBUF),
        compiler_params=pltpu.CompilerParams(use_tc_tiling_on_sc=False),
        name="gcn_row_agg",
    )
    def agg_kernel(table_hbm, ei_hbm, out_hbm, src_v, dst_v, rows_v, acc, *sems):
        cid = lax.axis_index("c")
        sid = lax.axis_index("s")
        cbase, cnt, w = _worker_range(cid, sid, gper, grem)
        base = sid * rpt

        @pl.when(w < grem)
        def _():
            pltpu.sync_copy(ei_hbm.at[0, pl.ds(cbase, cmax)], src_v)
            pltpu.sync_copy(ei_hbm.at[1, pl.ds(cbase, cmax)], dst_v)

        @pl.when(w >= grem)
        def _():
            pltpu.sync_copy(
                ei_hbm.at[0, pl.ds(cbase, gper * NBUF)],
                src_v.at[pl.ds(0, gper * NBUF)],
            )
            pltpu.sync_copy(
                ei_hbm.at[1, pl.ds(cbase, gper * NBUF)],
                dst_v.at[pl.ds(0, gper * NBUF)],
            )

        def zrow(i, carry):
            rows_v[0, i, :] = jnp.zeros((LANES,), jnp.float32)
            return carry

        lax.fori_loop(0, CH, zrow, 0)
        for t in range(rpt // CH):
            pltpu.sync_copy(rows_v.at[0], acc.at[pl.ds(base + t * CH, CH)])
        plsc.subcore_barrier()

        gsems = sems[:NBUF]
        ssems = sems[NBUF:]

        for b in range(NBUF):
            pltpu.async_copy(table_hbm.at[src_v.at[b]], rows_v.at[b], gsems[b])

        def group(g, carry):
            for b in range(NBUF):
                j = g * NBUF + b
                pltpu.make_async_copy(
                    table_hbm.at[src_v.at[j]], rows_v.at[b], gsems[b]
                ).wait()
                pltpu.async_copy(
                    rows_v.at[b], acc.at[dst_v.at[j]], ssems[b], add=True
                )
            for b in range(NBUF):
                j = g * NBUF + b
                pltpu.make_async_copy(
                    rows_v.at[b], acc.at[dst_v.at[j]], ssems[b]
                ).wait()

                @pl.when(j + NBUF < cnt)
                def _():
                    pltpu.async_copy(
                        table_hbm.at[src_v.at[j + NBUF]], rows_v.at[b], gsems[b]
                    )

            return carry

        lax.fori_loop(0, cnt // NBUF, group, 0)
        plsc.subcore_barrier()
        pltpu.sync_copy(
            acc.at[pl.ds(base, rpt)], out_hbm.at[cid, pl.ds(base, rpt)]
        )

    return agg_kernel


PK = 128 // D_HID


def _pack(v, rb):
    v3 = v.reshape(rb, PK, D_HID)
    return jnp.concatenate([v3[:, j, :] for j in range(PK)], axis=1)


def _unpack(p, rb):
    cols = [
        p[:, D_HID * j : D_HID * (j + 1)].reshape(rb, 1, D_HID) for j in range(PK)
    ]
    return jnp.concatenate(cols, axis=1).reshape(rb * PK, D_HID)


def _tc_a_body(x_ref, w_ref, degp_ref, zp_ref, d16_ref):
    rb = ROWBLK // PK
    z = jnp.dot(x_ref[...], w_ref[...], preferred_element_type=jnp.float32)
    deg8 = 1.0 + degp_ref[0] + degp_ref[1]
    dinv8 = 1.0 / jnp.sqrt(deg8)
    d16_p = jnp.concatenate(
        [jnp.broadcast_to(dinv8[:, j : j + 1], (rb, D_HID)) for j in range(PK)],
        axis=1,
    )
    d16_ref[...] = d16_p
    zp_ref[...] = _pack(z, rb) * d16_p


def _tc_a(x, W1, degp_v):
    n = x.shape[0]
    d_in = x.shape[1]
    grid = n // ROWBLK
    return pl.pallas_call(
        _tc_a_body,
        grid=(grid,),
        in_specs=[
            pl.BlockSpec((ROWBLK, d_in), lambda i: (i, 0)),
            pl.BlockSpec((d_in, D_HID), lambda i: (0, 0)),
            pl.BlockSpec((NC, ROWBLK // PK, PK), lambda i: (0, i, 0)),
        ],
        out_specs=[
            pl.BlockSpec((ROWBLK // PK, 128), lambda i: (i, 0)),
            pl.BlockSpec((ROWBLK // PK, 128), lambda i: (i, 0)),
        ],
        out_shape=[
            jax.ShapeDtypeStruct((n // PK, 128), jnp.float32),
            jax.ShapeDtypeStruct((n // PK, 128), jnp.float32),
        ],
    )(x, W1, degp_v)


def _tc_b_body(agg_ref, zp_ref, d16_ref, b1_ref, out_ref):
    s = agg_ref[0] + agg_ref[1] + zp_ref[...]
    d16 = d16_ref[...]
    b128 = jnp.concatenate([b1_ref[...]] * PK, axis=1)
    h1 = jnp.maximum(d16 * s + b128, 0.0)
    out_ref[...] = h1 * d16


def _tc_b(agg1_v, zp, d16, b1):
    rp = zp.shape[0]
    rb = ROWBLK // PK
    grid = rp // rb
    return pl.pallas_call(
        _tc_b_body,
        grid=(grid,),
        in_specs=[
            pl.BlockSpec((NC, rb, 128), lambda i: (0, i, 0)),
            pl.BlockSpec((rb, 128), lambda i: (i, 0)),
            pl.BlockSpec((rb, 128), lambda i: (i, 0)),
            pl.BlockSpec((1, D_HID), lambda i: (0, 0)),
        ],
        out_specs=pl.BlockSpec((rb, 128), lambda i: (i, 0)),
        out_shape=jax.ShapeDtypeStruct((rp, 128), jnp.float32),
    )(agg1_v, zp, d16, b1)


def _tc_c_body(agg_ref, h1p_ref, d16_ref, w2_ref, b2_ref, out_ref):
    s = agg_ref[0] + agg_ref[1] + h1p_ref[...]
    pre = _unpack(d16_ref[...] * s, ROWBLK // PK)
    h2 = jnp.dot(pre, w2_ref[...], preferred_element_type=jnp.float32)
    h2 = h2 + b2_ref[...]
    m = jnp.max(h2, axis=1, keepdims=True)
    e = jnp.exp(h2 - m)
    lse = jnp.log(jnp.sum(e, axis=1, keepdims=True))
    out_ref[...] = h2 - m - lse


def _tc_c(agg2_v, h1p, d16, W2, b2):
    rp = h1p.shape[0]
    n_cls = W2.shape[1]
    rb = ROWBLK // PK
    grid = rp // rb
    return pl.pallas_call(
        _tc_c_body,
        grid=(grid,),
        in_specs=[
            pl.BlockSpec((NC, rb, 128), lambda i: (0, i, 0)),
            pl.BlockSpec((rb, 128), lambda i: (i, 0)),
            pl.BlockSpec((rb, 128), lambda i: (i, 0)),
            pl.BlockSpec((D_HID, n_cls), lambda i: (0, 0)),
            pl.BlockSpec((1, n_cls), lambda i: (0, 0)),
        ],
        out_specs=pl.BlockSpec((ROWBLK, n_cls), lambda i: (i, 0)),
        out_shape=jax.ShapeDtypeStruct((rp * PK, n_cls), jnp.float32),
    )(agg2_v, h1p, d16, W2, b2)


def kernel(x, edge_index, W1, b1, W2, b2):
    n = x.shape[0]
    e = edge_index.shape[1]

    gtot = -(-e // (CH * NBUF))
    e_pad = gtot * CH * NBUF
    ei = edge_index.astype(jnp.int32)
    if e_pad != e:
        fill = jnp.stack(
            [
                jnp.zeros((e_pad - e,), jnp.int32),
                jnp.full((e_pad - e,), n, jnp.int32),
            ]
        )
        ei = jnp.concatenate([ei, fill], axis=1)
    ei3 = ei.reshape(2, e_pad // CH, CH)

    gper, grem = gtot // NW, gtot % NW
    n_pad = -(-(n + 1) // (NS * CH)) * (NS * CH)
    rpt = n_pad // NS

    degp = _make_deg_kernel(gper, grem, n_pad, rpt)(ei3)

    x_pad = jnp.pad(x, ((0, n_pad - n), (0, 0)))
    zp, d16 = _tc_a(x_pad, W1, degp.reshape(NC, n_pad // PK, PK))
    agg = _make_agg_kernel(gper, grem, n_pad, rpt)
    agg1 = agg(zp.reshape(n_pad, D_HID), ei3)
    h1p = _tc_b(agg1.reshape(NC, n_pad // PK, 128), zp, d16, b1.reshape(1, D_HID))
    agg2 = agg(h1p.reshape(n_pad, D_HID), ei3)
    out = _tc_c(
        agg2.reshape(NC, n_pad // PK, 128), h1p, d16, W2, b2.reshape(1, -1)
    )
    return out[:n]


# speedup vs baseline: 2.3847x; 1.0599x over previous
import functools

import jax
import jax.numpy as jnp
from jax import lax
from jax.experimental import pallas as pl
from jax.experimental.pallas import tpu as pltpu
from jax.experimental.pallas import tpu_sc as plsc

NC = 2
NS = 16
NW = NC * NS
LANES = 16
CH = 128
NBUF = 4
D_HID = 16
ROWBLK = 1024


def _mesh():
    return plsc.VectorSubcoreMesh(
        core_axis_name="c", subcore_axis_name="s", num_cores=NC, num_subcores=NS
    )


def _worker_range(cid, sid, gper, grem):
    w = sid * NC + cid
    extra = (w < grem).astype(jnp.int32)
    cnt = (gper + extra) * NBUF
    base = (gper * w + jnp.minimum(w, grem)) * NBUF
    return base, cnt, w


def _make_deg_kernel(gper, grem, n_pad, rpt):
    cmax = (gper + (1 if grem else 0)) * NBUF

    @functools.partial(
        pl.kernel,
        out_type=jax.ShapeDtypeStruct((NC, n_pad), jnp.float32),
        mesh=_mesh(),
        scratch_types=[
            pltpu.VMEM((cmax, CH), jnp.int32),
            pltpu.VMEM((CH,), jnp.float32),
            pltpu.VMEM((CH,), jnp.float32),
            pltpu.VMEM_SHARED((n_pad,), jnp.float32),
        ]
        + [pltpu.SemaphoreType.DMA] * NBUF,
        compiler_params=pltpu.CompilerParams(use_tc_tiling_on_sc=False),
        name="gcn_deg",
    )
    def deg_kernel(ei_hbm, out_hbm, idx_v, ones_v, zero_v, acc, *sems):
        cid = lax.axis_index("c")
        sid = lax.axis_index("s")
        cbase, cnt, w = _worker_range(cid, sid, gper, grem)
        for i in range(CH // LANES):
            ones_v[pl.ds(LANES * i, LANES)] = jnp.full((LANES,), 1.0, jnp.float32)
            zero_v[pl.ds(LANES * i, LANES)] = jnp.zeros((LANES,), jnp.float32)

        @pl.when(w < grem)
        def _():
            pltpu.sync_copy(ei_hbm.at[1, pl.ds(cbase, cmax)], idx_v)

        @pl.when(w >= grem)
        def _():
            pltpu.sync_copy(
                ei_hbm.at[1, pl.ds(cbase, gper * NBUF)],
                idx_v.at[pl.ds(0, gper * NBUF)],
            )

        base = sid * rpt
        for t in range(rpt // CH):
            pltpu.sync_copy(zero_v, acc.at[pl.ds(base + t * CH, CH)])
        plsc.subcore_barrier()

        for b in range(NBUF):
            pltpu.async_copy(ones_v, acc.at[idx_v.at[b]], sems[b], add=True)

        def group(g, carry):
            for b in range(NBUF):
                j = g * NBUF + b
                pltpu.make_async_copy(ones_v, acc.at[idx_v.at[j]], sems[b]).wait()

                @pl.when(j + NBUF < cnt)
                def _():
                    pltpu.async_copy(
                        ones_v, acc.at[idx_v.at[j + NBUF]], sems[b], add=True
                    )

            return carry

        lax.fori_loop(0, cnt // NBUF, group, 0)
        plsc.subcore_barrier()
        pltpu.sync_copy(acc.at[pl.ds(base, rpt)], out_hbm.at[cid, pl.ds(base, rpt)])

    return deg_kernel


def _make_agg_kernel(gper, grem, n_pad, rpt):
    cmax = (gper + (1 if grem else 0)) * NBUF

    @functools.partial(
        pl.kernel,
        out_type=jax.ShapeDtypeStruct((NC, n_pad, D_HID), jnp.float32),
        mesh=_mesh(),
        scratch_types=[
            pltpu.VMEM((cmax, CH), jnp.int32),
            pltpu.VMEM((cmax, CH), jnp.int32),
            pltpu.VMEM((NBUF, CH, D_HID), jnp.float32),
            pltpu.VMEM_SHARED((n_pad, D_HID), jnp.float32),
            pltpu.VMEM_SHARED((n_pad, D_HID), jnp.float32),
        ]
        + [pltpu.SemaphoreType.DMA] * (2 * NBUF),
        compiler_params=pltpu.CompilerParams(use_tc_tiling_on_sc=False),
        name="gcn_row_agg",
    )
    def agg_kernel(table_hbm, ei_hbm, out_hbm, src_v, dst_v, rows_v, acc,
                   table_sh, *sems):
        cid = lax.axis_index("c")
        sid = lax.axis_index("s")
        cbase, cnt, w = _worker_range(cid, sid, gper, grem)
        base = sid * rpt

        @pl.when(w < grem)
        def _():
            pltpu.sync_copy(ei_hbm.at[0, pl.ds(cbase, cmax)], src_v)
            pltpu.sync_copy(ei_hbm.at[1, pl.ds(cbase, cmax)], dst_v)

        @pl.when(w >= grem)
        def _():
            pltpu.sync_copy(
                ei_hbm.at[0, pl.ds(cbase, gper * NBUF)],
                src_v.at[pl.ds(0, gper * NBUF)],
            )
            pltpu.sync_copy(
                ei_hbm.at[1, pl.ds(cbase, gper * NBUF)],
                dst_v.at[pl.ds(0, gper * NBUF)],
            )

        def zrow(i, carry):
            rows_v[0, i, :] = jnp.zeros((LANES,), jnp.float32)
            return carry

        lax.fori_loop(0, CH, zrow, 0)
        for t in range(rpt // CH):
            pltpu.sync_copy(rows_v.at[0], acc.at[pl.ds(base + t * CH, CH)])
        pltpu.sync_copy(
            table_hbm.at[pl.ds(base, rpt)], table_sh.at[pl.ds(base, rpt)]
        )
        plsc.subcore_barrier()

        gsems = sems[:NBUF]
        ssems = sems[NBUF:]

        for b in range(NBUF):
            pltpu.async_copy(table_sh.at[src_v.at[b]], rows_v.at[b], gsems[b])

        def group(g, carry):
            for b in range(NBUF):
                j = g * NBUF + b
                pltpu.make_async_copy(
                    table_sh.at[src_v.at[j]], rows_v.at[b], gsems[b]
                ).wait()
                pltpu.async_copy(
                    rows_v.at[b], acc.at[dst_v.at[j]], ssems[b], add=True
                )
            for b in range(NBUF):
                j = g * NBUF + b
                pltpu.make_async_copy(
                    rows_v.at[b], acc.at[dst_v.at[j]], ssems[b]
                ).wait()

                @pl.when(j + NBUF < cnt)
                def _():
                    pltpu.async_copy(
                        table_sh.at[src_v.at[j + NBUF]], rows_v.at[b], gsems[b]
                    )

            return carry

        lax.fori_loop(0, cnt // NBUF, group, 0)
        plsc.subcore_barrier()
        pltpu.sync_copy(
            acc.at[pl.ds(base, rpt)], out_hbm.at[cid, pl.ds(base, rpt)]
        )

    return agg_kernel


PK = 128 // D_HID


def _pack(v, rb):
    v3 = v.reshape(rb, PK, D_HID)
    return jnp.concatenate([v3[:, j, :] for j in range(PK)], axis=1)


def _unpack(p, rb):
    cols = [
        p[:, D_HID * j : D_HID * (j + 1)].reshape(rb, 1, D_HID) for j in range(PK)
    ]
    return jnp.concatenate(cols, axis=1).reshape(rb * PK, D_HID)


def _tc_a_body(x_ref, w_ref, degp_ref, zp_ref, d16_ref):
    rb = ROWBLK // PK
    z = jnp.dot(x_ref[...], w_ref[...], preferred_element_type=jnp.float32)
    deg8 = 1.0 + degp_ref[0] + degp_ref[1]
    dinv8 = 1.0 / jnp.sqrt(deg8)
    d16_p = jnp.concatenate(
        [jnp.broadcast_to(dinv8[:, j : j + 1], (rb, D_HID)) for j in range(PK)],
        axis=1,
    )
    d16_ref[...] = d16_p
    zp_ref[...] = _pack(z, rb) * d16_p


def _tc_a(x, W1, degp_v):
    n = x.shape[0]
    d_in = x.shape[1]
    grid = n // ROWBLK
    return pl.pallas_call(
        _tc_a_body,
        grid=(grid,),
        in_specs=[
            pl.BlockSpec((ROWBLK, d_in), lambda i: (i, 0)),
            pl.BlockSpec((d_in, D_HID), lambda i: (0, 0)),
            pl.BlockSpec((NC, ROWBLK // PK, PK), lambda i: (0, i, 0)),
        ],
        out_specs=[
            pl.BlockSpec((ROWBLK // PK, 128), lambda i: (i, 0)),
            pl.BlockSpec((ROWBLK // PK, 128), lambda i: (i, 0)),
        ],
        out_shape=[
            jax.ShapeDtypeStruct((n // PK, 128), jnp.float32),
            jax.ShapeDtypeStruct((n // PK, 128), jnp.float32),
        ],
    )(x, W1, degp_v)


def _tc_b_body(agg_ref, zp_ref, d16_ref, b1_ref, out_ref):
    s = agg_ref[0] + agg_ref[1] + zp_ref[...]
    d16 = d16_ref[...]
    b128 = jnp.concatenate([b1_ref[...]] * PK, axis=1)
    h1 = jnp.maximum(d16 * s + b128, 0.0)
    out_ref[...] = h1 * d16


def _tc_b(agg1_v, zp, d16, b1):
    rp = zp.shape[0]
    rb = ROWBLK // PK
    grid = rp // rb
    return pl.pallas_call(
        _tc_b_body,
        grid=(grid,),
        in_specs=[
            pl.BlockSpec((NC, rb, 128), lambda i: (0, i, 0)),
            pl.BlockSpec((rb, 128), lambda i: (i, 0)),
            pl.BlockSpec((rb, 128), lambda i: (i, 0)),
            pl.BlockSpec((1, D_HID), lambda i: (0, 0)),
        ],
        out_specs=pl.BlockSpec((rb, 128), lambda i: (i, 0)),
        out_shape=jax.ShapeDtypeStruct((rp, 128), jnp.float32),
    )(agg1_v, zp, d16, b1)


def _tc_c_body(agg_ref, h1p_ref, d16_ref, w2_ref, b2_ref, out_ref):
    s = agg_ref[0] + agg_ref[1] + h1p_ref[...]
    pre = _unpack(d16_ref[...] * s, ROWBLK // PK)
    h2 = jnp.dot(pre, w2_ref[...], preferred_element_type=jnp.float32)
    h2 = h2 + b2_ref[...]
    m = jnp.max(h2, axis=1, keepdims=True)
    e = jnp.exp(h2 - m)
    lse = jnp.log(jnp.sum(e, axis=1, keepdims=True))
    out_ref[...] = h2 - m - lse


def _tc_c(agg2_v, h1p, d16, W2, b2, n):
    rp = h1p.shape[0]
    n_cls = W2.shape[1]
    rb = ROWBLK // PK
    grid = rp // rb
    return pl.pallas_call(
        _tc_c_body,
        grid=(grid,),
        in_specs=[
            pl.BlockSpec((NC, rb, 128), lambda i: (0, i, 0)),
            pl.BlockSpec((rb, 128), lambda i: (i, 0)),
            pl.BlockSpec((rb, 128), lambda i: (i, 0)),
            pl.BlockSpec((D_HID, n_cls), lambda i: (0, 0)),
            pl.BlockSpec((1, n_cls), lambda i: (0, 0)),
        ],
        out_specs=pl.BlockSpec((ROWBLK, n_cls), lambda i: (i, 0)),
        out_shape=jax.ShapeDtypeStruct((n, n_cls), jnp.float32),
    )(agg2_v, h1p, d16, W2, b2)


def kernel(x, edge_index, W1, b1, W2, b2):
    n = x.shape[0]
    e = edge_index.shape[1]

    gtot = -(-e // (CH * NBUF))
    e_pad = gtot * CH * NBUF
    ei = edge_index.astype(jnp.int32)
    if e_pad != e:
        fill = jnp.stack(
            [
                jnp.zeros((e_pad - e,), jnp.int32),
                jnp.full((e_pad - e,), n, jnp.int32),
            ]
        )
        ei = jnp.concatenate([ei, fill], axis=1)
    ei3 = ei.reshape(2, e_pad // CH, CH)

    gper, grem = gtot // NW, gtot % NW
    n_pad = -(-(n + 1) // (NS * CH)) * (NS * CH)
    rpt = n_pad // NS

    degp = _make_deg_kernel(gper, grem, n_pad, rpt)(ei3)

    x_pad = jnp.pad(x, ((0, n_pad - n), (0, 0)))
    zp, d16 = _tc_a(x_pad, W1, degp.reshape(NC, n_pad // PK, PK))
    agg = _make_agg_kernel(gper, grem, n_pad, rpt)
    agg1 = agg(zp.reshape(n_pad, D_HID), ei3)
    h1p = _tc_b(agg1.reshape(NC, n_pad // PK, 128), zp, d16, b1.reshape(1, D_HID))
    agg2 = agg(h1p.reshape(n_pad, D_HID), ei3)
    return _tc_c(
        agg2.reshape(NC, n_pad // PK, 128), h1p, d16, W2, b2.reshape(1, -1), n
    )


# speedup vs baseline: 2.3865x; 1.0007x over previous
import functools

import jax
import jax.numpy as jnp
from jax import lax
from jax.experimental import pallas as pl
from jax.experimental.pallas import tpu as pltpu
from jax.experimental.pallas import tpu_sc as plsc

NC = 2
NS = 16
NW = NC * NS
LANES = 16
CH = 128
NBUF = 4
D_HID = 16
ROWBLK = 1024


def _mesh():
    return plsc.VectorSubcoreMesh(
        core_axis_name="c", subcore_axis_name="s", num_cores=NC, num_subcores=NS
    )


def _worker_range(cid, sid, gper, grem):
    w = sid * NC + cid
    extra = (w < grem).astype(jnp.int32)
    cnt = (gper + extra) * NBUF
    base = (gper * w + jnp.minimum(w, grem)) * NBUF
    return base, cnt, w


def _make_deg_kernel(gper, grem, n_pad, rpt):
    cmax = (gper + (1 if grem else 0)) * NBUF

    @functools.partial(
        pl.kernel,
        out_type=jax.ShapeDtypeStruct((NC, n_pad), jnp.float32),
        mesh=_mesh(),
        scratch_types=[
            pltpu.VMEM((cmax, CH), jnp.int32),
            pltpu.VMEM((CH,), jnp.float32),
            pltpu.VMEM((CH,), jnp.float32),
            pltpu.VMEM_SHARED((n_pad,), jnp.float32),
        ]
        + [pltpu.SemaphoreType.DMA] * NBUF,
        compiler_params=pltpu.CompilerParams(use_tc_tiling_on_sc=False),
        name="gcn_deg",
    )
    def deg_kernel(ei_hbm, out_hbm, idx_v, ones_v, zero_v, acc, *sems):
        cid = lax.axis_index("c")
        sid = lax.axis_index("s")
        cbase, cnt, w = _worker_range(cid, sid, gper, grem)
        for i in range(CH // LANES):
            ones_v[pl.ds(LANES * i, LANES)] = jnp.full((LANES,), 1.0, jnp.float32)
            zero_v[pl.ds(LANES * i, LANES)] = jnp.zeros((LANES,), jnp.float32)

        @pl.when(w < grem)
        def _():
            pltpu.sync_copy(ei_hbm.at[1, pl.ds(cbase, cmax)], idx_v)

        @pl.when(w >= grem)
        def _():
            pltpu.sync_copy(
                ei_hbm.at[1, pl.ds(cbase, gper * NBUF)],
                idx_v.at[pl.ds(0, gper * NBUF)],
            )

        base = sid * rpt
        for t in range(rpt // CH):
            pltpu.sync_copy(zero_v, acc.at[pl.ds(base + t * CH, CH)])
        plsc.subcore_barrier()

        for b in range(NBUF):
            pltpu.async_copy(ones_v, acc.at[idx_v.at[b]], sems[b], add=True)

        def group(g, carry):
            for b in range(NBUF):
                j = g * NBUF + b
                pltpu.make_async_copy(ones_v, acc.at[idx_v.at[j]], sems[b]).wait()

                @pl.when(j + NBUF < cnt)
                def _():
                    pltpu.async_copy(
                        ones_v, acc.at[idx_v.at[j + NBUF]], sems[b], add=True
                    )

            return carry

        lax.fori_loop(0, cnt // NBUF, group, 0)
        plsc.subcore_barrier()
        pltpu.sync_copy(acc.at[pl.ds(base, rpt)], out_hbm.at[cid, pl.ds(base, rpt)])

    return deg_kernel


def _make_agg_kernel(gper, grem, n_pad, rpt):
    cmax = (gper + (1 if grem else 0)) * NBUF

    @functools.partial(
        pl.kernel,
        out_type=jax.ShapeDtypeStruct((NC, n_pad, D_HID), jnp.float32),
        mesh=_mesh(),
        scratch_types=[
            pltpu.VMEM((cmax, CH), jnp.int32),
            pltpu.VMEM((cmax, CH), jnp.int32),
            pltpu.VMEM((NBUF, CH, D_HID), jnp.float32),
            pltpu.VMEM_SHARED((n_pad, D_HID), jnp.float32),
            pltpu.VMEM_SHARED((n_pad, D_HID), jnp.float32),
        ]
        + [pltpu.SemaphoreType.DMA] * (2 * NBUF),
        compiler_params=pltpu.CompilerParams(use_tc_tiling_on_sc=False),
        name="gcn_row_agg",
    )
    def agg_kernel(table_hbm, ei_hbm, out_hbm, src_v, dst_v, rows_v, acc,
                   table_sh, *sems):
        cid = lax.axis_index("c")
        sid = lax.axis_index("s")
        cbase, cnt, w = _worker_range(cid, sid, gper, grem)
        base = sid * rpt

        @pl.when(w < grem)
        def _():
            pltpu.sync_copy(ei_hbm.at[0, pl.ds(cbase, cmax)], src_v)
            pltpu.sync_copy(ei_hbm.at[1, pl.ds(cbase, cmax)], dst_v)

        @pl.when(w >= grem)
        def _():
            pltpu.sync_copy(
                ei_hbm.at[0, pl.ds(cbase, gper * NBUF)],
                src_v.at[pl.ds(0, gper * NBUF)],
            )
            pltpu.sync_copy(
                ei_hbm.at[1, pl.ds(cbase, gper * NBUF)],
                dst_v.at[pl.ds(0, gper * NBUF)],
            )

        def zrow(i, carry):
            rows_v[0, i, :] = jnp.zeros((LANES,), jnp.float32)
            return carry

        lax.fori_loop(0, CH, zrow, 0)
        for t in range(rpt // CH):
            pltpu.sync_copy(rows_v.at[0], acc.at[pl.ds(base + t * CH, CH)])
        pltpu.sync_copy(
            table_hbm.at[pl.ds(base, rpt)], table_sh.at[pl.ds(base, rpt)]
        )
        plsc.subcore_barrier()

        gsems = sems[:NBUF]
        ssems = sems[NBUF:]

        for b in range(NBUF):
            pltpu.async_copy(table_sh.at[src_v.at[b]], rows_v.at[b], gsems[b])

        def group(g, carry):
            for b in range(NBUF):
                j = g * NBUF + b
                pltpu.make_async_copy(
                    table_sh.at[src_v.at[j]], rows_v.at[b], gsems[b]
                ).wait()
                pltpu.async_copy(
                    rows_v.at[b], acc.at[dst_v.at[j]], ssems[b], add=True
                )
            for b in range(NBUF):
                j = g * NBUF + b
                pltpu.make_async_copy(
                    rows_v.at[b], acc.at[dst_v.at[j]], ssems[b]
                ).wait()

                @pl.when(j + NBUF < cnt)
                def _():
                    pltpu.async_copy(
                        table_sh.at[src_v.at[j + NBUF]], rows_v.at[b], gsems[b]
                    )

            return carry

        lax.fori_loop(0, cnt // NBUF, group, 0)
        plsc.subcore_barrier()
        pltpu.sync_copy(
            acc.at[pl.ds(base, rpt)], out_hbm.at[cid, pl.ds(base, rpt)]
        )

    return agg_kernel


PK = 128 // D_HID


def _pack(v, rb):
    v3 = v.reshape(rb, PK, D_HID)
    return jnp.concatenate([v3[:, j, :] for j in range(PK)], axis=1)


def _unpack(p, rb, d=D_HID):
    cols = [p[:, d * j : d * (j + 1)].reshape(rb, 1, d) for j in range(PK)]
    return jnp.concatenate(cols, axis=1).reshape(rb * PK, d)


def _tc_a_body(x_ref, w_ref, degp_ref, zp_ref, d16_ref):
    rb = ROWBLK // PK
    z = jnp.dot(x_ref[...], w_ref[...], preferred_element_type=jnp.float32)
    deg8 = 1.0 + degp_ref[0] + degp_ref[1]
    dinv8 = 1.0 / jnp.sqrt(deg8)
    d16_p = jnp.concatenate(
        [jnp.broadcast_to(dinv8[:, j : j + 1], (rb, D_HID)) for j in range(PK)],
        axis=1,
    )
    d16_ref[...] = d16_p
    zp_ref[...] = _pack(z, rb) * d16_p


def _tc_a(x, W1, degp_v):
    n = x.shape[0]
    d_in = x.shape[1]
    grid = n // ROWBLK
    return pl.pallas_call(
        _tc_a_body,
        grid=(grid,),
        in_specs=[
            pl.BlockSpec((ROWBLK, d_in), lambda i: (i, 0)),
            pl.BlockSpec((d_in, D_HID), lambda i: (0, 0)),
            pl.BlockSpec((NC, ROWBLK // PK, PK), lambda i: (0, i, 0)),
        ],
        out_specs=[
            pl.BlockSpec((ROWBLK // PK, 128), lambda i: (i, 0)),
            pl.BlockSpec((ROWBLK // PK, 128), lambda i: (i, 0)),
        ],
        out_shape=[
            jax.ShapeDtypeStruct((n // PK, 128), jnp.float32),
            jax.ShapeDtypeStruct((n // PK, 128), jnp.float32),
        ],
    )(x, W1, degp_v)


def _tc_b_body(agg_ref, zp_ref, d16_ref, b1_ref, out_ref):
    s = agg_ref[0] + agg_ref[1] + zp_ref[...]
    d16 = d16_ref[...]
    b128 = jnp.concatenate([b1_ref[...]] * PK, axis=1)
    h1 = jnp.maximum(d16 * s + b128, 0.0)
    out_ref[...] = h1 * d16


def _tc_b(agg1_v, zp, d16, b1):
    rp = zp.shape[0]
    rb = ROWBLK // PK
    grid = rp // rb
    return pl.pallas_call(
        _tc_b_body,
        grid=(grid,),
        in_specs=[
            pl.BlockSpec((NC, rb, 128), lambda i: (0, i, 0)),
            pl.BlockSpec((rb, 128), lambda i: (i, 0)),
            pl.BlockSpec((rb, 128), lambda i: (i, 0)),
            pl.BlockSpec((1, D_HID), lambda i: (0, 0)),
        ],
        out_specs=pl.BlockSpec((rb, 128), lambda i: (i, 0)),
        out_shape=jax.ShapeDtypeStruct((rp, 128), jnp.float32),
    )(agg1_v, zp, d16, b1)


def _tc_c_body(agg_ref, h1p_ref, d16_ref, w2b_ref, sb_ref, b2_ref, out_ref):
    rb = ROWBLK // PK
    n_cls = b2_ref.shape[1]
    s = agg_ref[0] + agg_ref[1] + h1p_ref[...]
    pre_p = d16_ref[...] * s
    h2 = jnp.dot(pre_p, w2b_ref[...], preferred_element_type=jnp.float32)
    h2 = h2 + jnp.concatenate([b2_ref[...]] * PK, axis=1)
    m = jnp.max(h2, axis=1, keepdims=True)
    e = jnp.exp(h2 - m)
    gs = jnp.dot(e, sb_ref[...], preferred_element_type=jnp.float32)
    o_pk = h2 - m - jnp.log(gs)
    out_ref[...] = _unpack(o_pk, rb, n_cls)


def _tc_c(agg2_v, h1p, d16, W2blk, Sblk, b2, n):
    rp = h1p.shape[0]
    n_cls = b2.shape[1]
    rb = ROWBLK // PK
    grid = rp // rb
    return pl.pallas_call(
        _tc_c_body,
        grid=(grid,),
        in_specs=[
            pl.BlockSpec((NC, rb, 128), lambda i: (0, i, 0)),
            pl.BlockSpec((rb, 128), lambda i: (i, 0)),
            pl.BlockSpec((rb, 128), lambda i: (i, 0)),
            pl.BlockSpec((128, PK * n_cls), lambda i: (0, 0)),
            pl.BlockSpec((PK * n_cls, PK * n_cls), lambda i: (0, 0)),
            pl.BlockSpec((1, n_cls), lambda i: (0, 0)),
        ],
        out_specs=pl.BlockSpec((ROWBLK, n_cls), lambda i: (i, 0)),
        out_shape=jax.ShapeDtypeStruct((n, n_cls), jnp.float32),
    )(agg2_v, h1p, d16, W2blk, Sblk, b2)


def kernel(x, edge_index, W1, b1, W2, b2):
    n = x.shape[0]
    e = edge_index.shape[1]

    gtot = -(-e // (CH * NBUF))
    e_pad = gtot * CH * NBUF
    ei = edge_index.astype(jnp.int32)
    if e_pad != e:
        fill = jnp.stack(
            [
                jnp.zeros((e_pad - e,), jnp.int32),
                jnp.full((e_pad - e,), n, jnp.int32),
            ]
        )
        ei = jnp.concatenate([ei, fill], axis=1)
    ei3 = ei.reshape(2, e_pad // CH, CH)

    gper, grem = gtot // NW, gtot % NW
    n_pad = -(-(n + 1) // (NS * CH)) * (NS * CH)
    rpt = n_pad // NS

    degp = _make_deg_kernel(gper, grem, n_pad, rpt)(ei3)

    x_pad = jnp.pad(x, ((0, n_pad - n), (0, 0)))
    zp, d16 = _tc_a(x_pad, W1, degp.reshape(NC, n_pad // PK, PK))
    agg = _make_agg_kernel(gper, grem, n_pad, rpt)
    agg1 = agg(zp.reshape(n_pad, D_HID), ei3)
    h1p = _tc_b(agg1.reshape(NC, n_pad // PK, 128), zp, d16, b1.reshape(1, D_HID))
    agg2 = agg(h1p.reshape(n_pad, D_HID), ei3)
    n_cls = W2.shape[1]
    eye8 = jnp.eye(PK, dtype=jnp.float32)
    W2blk = jnp.kron(eye8, W2)
    Sblk = jnp.kron(eye8, jnp.ones((n_cls, n_cls), jnp.float32))
    return _tc_c(
        agg2.reshape(NC, n_pad // PK, 128), h1p, d16, W2blk, Sblk,
        b2.reshape(1, -1), n,
    )


# speedup vs baseline: 2.4191x; 1.0137x over previous
import functools

import jax
import jax.numpy as jnp
from jax import lax
from jax.experimental import pallas as pl
from jax.experimental.pallas import tpu as pltpu
from jax.experimental.pallas import tpu_sc as plsc

NC = 2
NS = 16
NW = NC * NS
LANES = 16
CH = 128
NBUF = 4
RING = 8
D_HID = 16
ROWBLK = 1024


def _mesh():
    return plsc.VectorSubcoreMesh(
        core_axis_name="c", subcore_axis_name="s", num_cores=NC, num_subcores=NS
    )


def _worker_range(cid, sid, gper, grem):
    w = sid * NC + cid
    extra = (w < grem).astype(jnp.int32)
    cnt = (gper + extra) * NBUF
    base = (gper * w + jnp.minimum(w, grem)) * NBUF
    return base, cnt, w


def _worker_group_range(cid, sid, gper, grem):
    w = sid * NC + cid
    extra = (w < grem).astype(jnp.int32)
    cnt = gper + extra
    base = gper * w + jnp.minimum(w, grem)
    return base, cnt, w


def _make_deg_kernel(gper, grem, n_pad, rpt):
    cmax = (gper + (1 if grem else 0)) * NBUF

    @functools.partial(
        pl.kernel,
        out_type=jax.ShapeDtypeStruct((NC, n_pad), jnp.float32),
        mesh=_mesh(),
        scratch_types=[
            pltpu.VMEM((cmax, CH), jnp.int32),
            pltpu.VMEM((CH,), jnp.float32),
            pltpu.VMEM((CH,), jnp.float32),
            pltpu.VMEM_SHARED((n_pad,), jnp.float32),
        ]
        + [pltpu.SemaphoreType.DMA] * NBUF,
        compiler_params=pltpu.CompilerParams(use_tc_tiling_on_sc=False),
        name="gcn_deg",
    )
    def deg_kernel(ei_hbm, out_hbm, idx_v, ones_v, zero_v, acc, *sems):
        cid = lax.axis_index("c")
        sid = lax.axis_index("s")
        cbase, cnt, w = _worker_range(cid, sid, gper, grem)
        for i in range(CH // LANES):
            ones_v[pl.ds(LANES * i, LANES)] = jnp.full((LANES,), 1.0, jnp.float32)
            zero_v[pl.ds(LANES * i, LANES)] = jnp.zeros((LANES,), jnp.float32)

        @pl.when(w < grem)
        def _():
            pltpu.sync_copy(ei_hbm.at[1, pl.ds(cbase, cmax)], idx_v)

        @pl.when(w >= grem)
        def _():
            pltpu.sync_copy(
                ei_hbm.at[1, pl.ds(cbase, gper * NBUF)],
                idx_v.at[pl.ds(0, gper * NBUF)],
            )

        base = sid * rpt
        for t in range(rpt // CH):
            pltpu.sync_copy(zero_v, acc.at[pl.ds(base + t * CH, CH)])
        plsc.subcore_barrier()

        for b in range(NBUF):
            pltpu.async_copy(ones_v, acc.at[idx_v.at[b]], sems[b], add=True)

        def group(g, carry):
            for b in range(NBUF):
                j = g * NBUF + b
                pltpu.make_async_copy(ones_v, acc.at[idx_v.at[j]], sems[b]).wait()

                @pl.when(j + NBUF < cnt)
                def _():
                    pltpu.async_copy(
                        ones_v, acc.at[idx_v.at[j + NBUF]], sems[b], add=True
                    )

            return carry

        lax.fori_loop(0, cnt // NBUF, group, 0)
        plsc.subcore_barrier()
        pltpu.sync_copy(acc.at[pl.ds(base, rpt)], out_hbm.at[cid, pl.ds(base, rpt)])

    return deg_kernel


def _make_agg_kernel(gper, grem, n_pad, rpt):
    cmax = (gper + (1 if grem else 0)) * NBUF

    @functools.partial(
        pl.kernel,
        out_type=jax.ShapeDtypeStruct((NC, n_pad, D_HID), jnp.float32),
        mesh=_mesh(),
        scratch_types=[
            pltpu.VMEM((cmax, CH), jnp.int32),
            pltpu.VMEM((cmax, CH), jnp.int32),
            pltpu.VMEM((RING, CH, D_HID), jnp.float32),
            pltpu.VMEM_SHARED((n_pad, D_HID), jnp.float32),
            pltpu.VMEM_SHARED((n_pad, D_HID), jnp.float32),
        ]
        + [pltpu.SemaphoreType.DMA] * (2 * RING),
        compiler_params=pltpu.CompilerParams(use_tc_tiling_on_sc=False),
        name="gcn_row_agg",
    )
    def agg_kernel(table_hbm, ei_hbm, out_hbm, src_v, dst_v, rows_v, acc,
                   table_sh, *sems):
        cid = lax.axis_index("c")
        sid = lax.axis_index("s")
        cbase, cnt, w = _worker_range(cid, sid, gper, grem)
        base = sid * rpt

        @pl.when(w < grem)
        def _():
            pltpu.sync_copy(ei_hbm.at[0, pl.ds(cbase, cmax)], src_v)
            pltpu.sync_copy(ei_hbm.at[1, pl.ds(cbase, cmax)], dst_v)

        @pl.when(w >= grem)
        def _():
            pltpu.sync_copy(
                ei_hbm.at[0, pl.ds(cbase, gper * NBUF)],
                src_v.at[pl.ds(0, gper * NBUF)],
            )
            pltpu.sync_copy(
                ei_hbm.at[1, pl.ds(cbase, gper * NBUF)],
                dst_v.at[pl.ds(0, gper * NBUF)],
            )

        def zrow(i, carry):
            rows_v[0, i, :] = jnp.zeros((LANES,), jnp.float32)
            return carry

        lax.fori_loop(0, CH, zrow, 0)
        for t in range(rpt // CH):
            pltpu.sync_copy(rows_v.at[0], acc.at[pl.ds(base + t * CH, CH)])
        pltpu.sync_copy(
            table_hbm.at[pl.ds(base, rpt)], table_sh.at[pl.ds(base, rpt)]
        )
        plsc.subcore_barrier()

        gsems = sems[:RING]
        ssems = sems[RING:]

        for b in range(RING):
            pltpu.async_copy(table_sh.at[src_v.at[b]], rows_v.at[b], gsems[b])

        csteps = -(-cmax // RING)

        def group(g, carry):
            for b in range(RING):
                j = g * RING + b

                @pl.when(j < cnt)
                def _():
                    pltpu.make_async_copy(
                        table_sh.at[src_v.at[j]], rows_v.at[b], gsems[b]
                    ).wait()
                    pltpu.async_copy(
                        rows_v.at[b], acc.at[dst_v.at[j]], ssems[b], add=True
                    )

            for b in range(RING):
                j = g * RING + b

                @pl.when(j < cnt)
                def _():
                    pltpu.make_async_copy(
                        rows_v.at[b], acc.at[dst_v.at[j]], ssems[b]
                    ).wait()

                    @pl.when(j + RING < cnt)
                    def _():
                        pltpu.async_copy(
                            table_sh.at[src_v.at[j + RING]], rows_v.at[b],
                            gsems[b],
                        )

            return carry

        lax.fori_loop(0, csteps, group, 0)
        plsc.subcore_barrier()
        pltpu.sync_copy(
            acc.at[pl.ds(base, rpt)], out_hbm.at[cid, pl.ds(base, rpt)]
        )

    return agg_kernel


PK = 128 // D_HID


def _pack(v, rb):
    v3 = v.reshape(rb, PK, D_HID)
    return jnp.concatenate([v3[:, j, :] for j in range(PK)], axis=1)


def _unpack(p, rb, d=D_HID):
    cols = [p[:, d * j : d * (j + 1)].reshape(rb, 1, d) for j in range(PK)]
    return jnp.concatenate(cols, axis=1).reshape(rb * PK, d)


def _tc_a_body(x_ref, w_ref, degp_ref, zp_ref, d16_ref):
    rb = ROWBLK // PK
    z = jnp.dot(x_ref[...], w_ref[...], preferred_element_type=jnp.float32)
    deg8 = 1.0 + degp_ref[0] + degp_ref[1]
    dinv8 = 1.0 / jnp.sqrt(deg8)
    d16_p = jnp.concatenate(
        [jnp.broadcast_to(dinv8[:, j : j + 1], (rb, D_HID)) for j in range(PK)],
        axis=1,
    )
    d16_ref[...] = d16_p
    zp_ref[...] = _pack(z, rb) * d16_p


def _tc_a(x, W1, degp_v):
    n = x.shape[0]
    d_in = x.shape[1]
    grid = n // ROWBLK
    return pl.pallas_call(
        _tc_a_body,
        grid=(grid,),
        in_specs=[
            pl.BlockSpec((ROWBLK, d_in), lambda i: (i, 0)),
            pl.BlockSpec((d_in, D_HID), lambda i: (0, 0)),
            pl.BlockSpec((NC, ROWBLK // PK, PK), lambda i: (0, i, 0)),
        ],
        out_specs=[
            pl.BlockSpec((ROWBLK // PK, 128), lambda i: (i, 0)),
            pl.BlockSpec((ROWBLK // PK, 128), lambda i: (i, 0)),
        ],
        out_shape=[
            jax.ShapeDtypeStruct((n // PK, 128), jnp.float32),
            jax.ShapeDtypeStruct((n // PK, 128), jnp.float32),
        ],
    )(x, W1, degp_v)


def _tc_b_body(agg_ref, zp_ref, d16_ref, b1_ref, out_ref):
    s = agg_ref[0] + agg_ref[1] + zp_ref[...]
    d16 = d16_ref[...]
    b128 = jnp.concatenate([b1_ref[...]] * PK, axis=1)
    h1 = jnp.maximum(d16 * s + b128, 0.0)
    out_ref[...] = h1 * d16


def _tc_b(agg1_v, zp, d16, b1):
    rp = zp.shape[0]
    rb = ROWBLK // PK
    grid = rp // rb
    return pl.pallas_call(
        _tc_b_body,
        grid=(grid,),
        in_specs=[
            pl.BlockSpec((NC, rb, 128), lambda i: (0, i, 0)),
            pl.BlockSpec((rb, 128), lambda i: (i, 0)),
            pl.BlockSpec((rb, 128), lambda i: (i, 0)),
            pl.BlockSpec((1, D_HID), lambda i: (0, 0)),
        ],
        out_specs=pl.BlockSpec((rb, 128), lambda i: (i, 0)),
        out_shape=jax.ShapeDtypeStruct((rp, 128), jnp.float32),
    )(agg1_v, zp, d16, b1)


def _tc_c_body(agg_ref, h1p_ref, d16_ref, w2b_ref, sb_ref, b2_ref, out_ref):
    rb = ROWBLK // PK
    n_cls = b2_ref.shape[1]
    s = agg_ref[0] + agg_ref[1] + h1p_ref[...]
    pre_p = d16_ref[...] * s
    h2 = jnp.dot(pre_p, w2b_ref[...], preferred_element_type=jnp.float32)
    h2 = h2 + jnp.concatenate([b2_ref[...]] * PK, axis=1)
    m = jnp.max(h2, axis=1, keepdims=True)
    e = jnp.exp(h2 - m)
    gs = jnp.dot(e, sb_ref[...], preferred_element_type=jnp.float32)
    o_pk = h2 - m - jnp.log(gs)
    out_ref[...] = _unpack(o_pk, rb, n_cls)


def _tc_c(agg2_v, h1p, d16, W2blk, Sblk, b2, n):
    rp = h1p.shape[0]
    n_cls = b2.shape[1]
    rb = ROWBLK // PK
    grid = rp // rb
    return pl.pallas_call(
        _tc_c_body,
        grid=(grid,),
        in_specs=[
            pl.BlockSpec((NC, rb, 128), lambda i: (0, i, 0)),
            pl.BlockSpec((rb, 128), lambda i: (i, 0)),
            pl.BlockSpec((rb, 128), lambda i: (i, 0)),
            pl.BlockSpec((128, PK * n_cls), lambda i: (0, 0)),
            pl.BlockSpec((PK * n_cls, PK * n_cls), lambda i: (0, 0)),
            pl.BlockSpec((1, n_cls), lambda i: (0, 0)),
        ],
        out_specs=pl.BlockSpec((ROWBLK, n_cls), lambda i: (i, 0)),
        out_shape=jax.ShapeDtypeStruct((n, n_cls), jnp.float32),
    )(agg2_v, h1p, d16, W2blk, Sblk, b2)


def kernel(x, edge_index, W1, b1, W2, b2):
    n = x.shape[0]
    e = edge_index.shape[1]

    gtot = -(-e // (CH * NBUF))
    e_pad = gtot * CH * NBUF
    ei = edge_index.astype(jnp.int32)
    if e_pad != e:
        fill = jnp.stack(
            [
                jnp.zeros((e_pad - e,), jnp.int32),
                jnp.full((e_pad - e,), n, jnp.int32),
            ]
        )
        ei = jnp.concatenate([ei, fill], axis=1)
    ei3 = ei.reshape(2, e_pad // CH, CH)

    gper, grem = gtot // NW, gtot % NW
    n_pad = -(-(n + 1) // (NS * CH)) * (NS * CH)
    rpt = n_pad // NS

    degp = _make_deg_kernel(gper, grem, n_pad, rpt)(ei3)

    x_pad = jnp.pad(x, ((0, n_pad - n), (0, 0)))
    zp, d16 = _tc_a(x_pad, W1, degp.reshape(NC, n_pad // PK, PK))
    agg = _make_agg_kernel(gper, grem, n_pad, rpt)
    agg1 = agg(zp.reshape(n_pad, D_HID), ei3)
    h1p = _tc_b(agg1.reshape(NC, n_pad // PK, 128), zp, d16, b1.reshape(1, D_HID))
    agg2 = agg(h1p.reshape(n_pad, D_HID), ei3)
    n_cls = W2.shape[1]
    eye8 = jnp.eye(PK, dtype=jnp.float32)
    W2blk = jnp.kron(eye8, W2)
    Sblk = jnp.kron(eye8, jnp.ones((n_cls, n_cls), jnp.float32))
    return _tc_c(
        agg2.reshape(NC, n_pad // PK, 128), h1p, d16, W2blk, Sblk,
        b2.reshape(1, -1), n,
    )
